# baseline (speedup 1.0000x reference)
import numpy as np
import jax
import jax.numpy as jnp
from jax import lax
from jax.experimental import pallas as pl
from jax.experimental.pallas import tpu as pltpu
from jax.experimental.pallas import tpu_sc as plsc

B = 4096
S = 200
D = 64
V = 1000000
NW = 32
BB = 128
L = 16
NSLAB = 25
CS = 256
NFULL = V // CS
TAIL0 = NFULL * CS
TAILW = V - TAIL0


def _positional_encoding_np():
    pos = np.arange(S, dtype=np.float32)[:, None]
    idx = np.arange(D, dtype=np.float32)[None, :]
    angle_rates = 1.0 / np.power(
        np.float32(10000.0), 2.0 * np.floor(idx / 2.0) / np.float32(D)
    )
    angle_rads = (pos * angle_rates).astype(np.float32)
    sines = np.sin(angle_rads[:, 0::2])
    cosines = np.cos(angle_rads[:, 1::2])
    pe = np.concatenate([sines[:, :, None], cosines[:, :, None]], axis=-1)
    return pe.reshape(S, D).astype(np.float32)


_PE = _positional_encoding_np()
_PE_DIAG = np.empty((S, D, L), np.float32)
for _k in range(D // L):
    for _d in range(L):
        for _l in range(L):
            _PE_DIAG[:, L * _k + _d, _l] = _PE[:, L * _k + (_d + _l) % L]


def _relayout_body(tt_hbm, t8_hbm, i0, i1, o0, o1, it, is0, is1, os0, os1):
    wid = lax.axis_index("s") * 2 + lax.axis_index("c")
    iota = lax.iota(jnp.int32, L)

    def in_start(s, ibuf, isem):
        c0 = pl.multiple_of(s * CS, CS)
        pltpu.make_async_copy(tt_hbm.at[:, pl.ds(c0, CS)], ibuf, isem).start()

    def in_wait(ibuf, isem):
        pltpu.make_async_copy(tt_hbm.at[:, pl.ds(0, CS)], ibuf, isem).wait()

    def out_start(s, obuf, osem):
        r0 = pl.multiple_of(s * CS, CS)
        pltpu.make_async_copy(obuf, t8_hbm.at[pl.ds(r0, CS)], osem).start()

    def out_wait(obuf, osem):
        pltpu.make_async_copy(obuf, t8_hbm.at[pl.ds(0, CS)], osem).wait()

    def transpose_slab(ibuf, obuf):
        rows = [iota + g * L for g in range(CS // L)]

        @plsc.parallel_loop(0, L, unroll=8)
        def d_body(d):
            diag = lax.bitwise_and(iota + d, L - 1)
            for k in range(D // L):
                kd = diag + k * L
                for g in range(CS // L):
                    val = plsc.load_gather(ibuf, [kd, rows[g]])
                    plsc.store_scatter(obuf, [rows[g], kd], val * 8.0)

    ibufs, obufs = (i0, i1), (o0, o1)
    isems, osems = (is0, is1), (os0, os1)

    nmine = jnp.int32(NFULL // NW) + jnp.where(wid < NFULL % NW, 1, 0)

    in_start(wid, i0, is0)
    in_start(wid + NW, i1, is1)

    def step(i, carry):
        for half in range(2):
            ibuf, obuf = ibufs[half], obufs[half]
            isem, osem = isems[half], osems[half]
            ii = i * 2 + half
            s = wid + ii * NW

            @pl.when(ii < nmine)
            def _():
                in_wait(ibuf, isem)

                @pl.when(ii >= 2)
                def _():
                    out_wait(obuf, osem)

                transpose_slab(ibuf, obuf)
                out_start(s, obuf, osem)

                @pl.when(ii + 2 < nmine)
                def _():
                    in_start(s + 2 * NW, ibuf, isem)

        return carry

    lax.fori_loop(0, (NFULL // NW + 2) // 2, step, 0)

    @pl.when(nmine >= 1)
    def _():
        out_wait(o0, os0)

    @pl.when(nmine >= 2)
    def _():
        out_wait(o1, os1)

    @pl.when(wid == 0)
    def _():
        pltpu.sync_copy(tt_hbm.at[:, pl.ds(TAIL0, TAILW)], it)
        rows = [lax.iota(jnp.int32, L) + g * L for g in range(TAILW // L)]

        @plsc.parallel_loop(0, L, unroll=8)
        def d_body(d):
            diag = lax.bitwise_and(lax.iota(jnp.int32, L) + d, L - 1)
            for k in range(D // L):
                kd = diag + k * L
                for g in range(TAILW // L):
                    val = plsc.load_gather(it, [kd, rows[g]])
                    plsc.store_scatter(o0, [rows[g], kd], val * 8.0)
        pltpu.sync_copy(o0.at[pl.ds(0, TAILW)], t8_hbm.at[pl.ds(TAIL0, TAILW)])


_relayout = pl.kernel(
    _relayout_body,
    out_type=jax.ShapeDtypeStruct((V, 2 * D), jnp.float32),
    mesh=plsc.VectorSubcoreMesh(core_axis_name="c", subcore_axis_name="s"),
    scratch_types=[
        pltpu.VMEM((D, CS), jnp.float32),
        pltpu.VMEM((D, CS), jnp.float32),
        pltpu.VMEM((CS, 2 * D), jnp.float32),
        pltpu.VMEM((CS, 2 * D), jnp.float32),
        pltpu.VMEM((D, TAILW), jnp.float32),
        pltpu.SemaphoreType.DMA,
        pltpu.SemaphoreType.DMA,
        pltpu.SemaphoreType.DMA,
        pltpu.SemaphoreType.DMA,
    ],
    compiler_params=pltpu.CompilerParams(
        use_tc_tiling_on_sc=True, needs_layout_passes=False
    ),
)


def _lookup_body(t8_hbm, x_hbm, pe_hbm, out_hbm,
                 ixraw, g0, g1, o0, o1, pe0, pe1,
                 ixs, gs0, gs1, os0, os1, ps0, ps1):
    wid = lax.axis_index("s") * 2 + lax.axis_index("c")
    sid0 = wid * NSLAB

    iota = lax.iota(jnp.int32, L)

    def slab_jb(k):
        sid = sid0 + k
        j0 = pl.multiple_of(lax.shift_left(lax.shift_right_logical(sid, 5), 3), 8)
        b0 = pl.multiple_of(lax.shift_left(lax.bitwise_and(sid, 31), 7), BB)
        return j0, b0

    def block_jb(n):
        j0, b0 = slab_jb(lax.shift_right_logical(n, 3))
        return j0 + lax.bitwise_and(n, 7), b0

    def slab_copy_start(k, q):
        j0, b0 = slab_jb(k)
        pltpu.make_async_copy(
            x_hbm.at[pl.ds(j0, 8), pl.ds(b0, BB)], ixraw.at[q], ixs.at[q]
        ).start()

    def slab_copy_wait(q):
        pltpu.make_async_copy(
            x_hbm.at[pl.ds(0, 8), pl.ds(0, BB)], ixraw.at[q], ixs.at[q]
        ).wait()

    def gather_start(gbuf, gsem, q, r):
        pltpu.make_async_copy(t8_hbm.at[ixraw.at[q, r]], gbuf, gsem).start()

    def gather_wait(gbuf, gsem):
        pltpu.make_async_copy(t8_hbm.at[ixraw.at[0, 0]], gbuf, gsem).wait()

    def pe_start(n, pebuf, pesem):
        j, _ = block_jb(n)
        pltpu.make_async_copy(pe_hbm.at[j], pebuf, pesem).start()

    def pe_wait(pebuf, pesem):
        pltpu.make_async_copy(pe_hbm.at[0], pebuf, pesem).wait()

    def out_start(n, obuf, osem):
        j, b0 = block_jb(n)
        pltpu.make_async_copy(
            obuf, out_hbm.at[j, :, pl.ds(b0, BB)], osem
        ).start()

    def out_wait(obuf, osem):
        pltpu.make_async_copy(
            obuf, out_hbm.at[0, :, pl.ds(0, BB)], osem
        ).wait()

    def compute(gbuf, obuf, pebuf):
        rows = [iota + g * L for g in range(BB // L)]

        @plsc.parallel_loop(0, L, unroll=8)
        def d_body(d):
            diag = lax.bitwise_and(iota + d, L - 1)
            for k in range(D // L):
                kd = diag + k * L
                pev = pebuf[k * L + d, :]
                for g in range(BB // L):
                    val = plsc.load_gather(gbuf, [rows[g], kd])
                    plsc.store_scatter(obuf, [kd, rows[g]], val + pev)

    gbufs, obufs, gsems, osems = (g0, g1), (o0, o1), (gs0, gs1), (os0, os1)
    pebufs, pesems = (pe0, pe1), (ps0, ps1)

    def block(n, p):
        gbuf, obuf, gsem, osem = gbufs[p], obufs[p], gsems[p], osems[p]
        pebuf, pesem = pebufs[p], pesems[p]
        k = lax.shift_right_logical(n, 3)
        r = lax.bitwise_and(n, 7)
        qnext = lax.bitwise_and(k + 1, 1)

        @pl.when(jnp.logical_and(r == 0, k + 1 < NSLAB))
        def _():
            slab_copy_start(k + 1, qnext)

        gather_wait(gbuf, gsem)
        pe_wait(pebuf, pesem)

        @pl.when(n >= 2)
        def _():
            out_wait(obuf, osem)

        compute(gbuf, obuf, pebuf)
        out_start(n, obuf, osem)

        @pl.when(jnp.logical_and(r == 6, k + 1 < NSLAB))
        def _():
            slab_copy_wait(qnext)

        @pl.when(n + 2 < 8 * NSLAB)
        def _():
            n2 = n + 2
            q2 = lax.bitwise_and(lax.shift_right_logical(n2, 3), 1)
            r2 = lax.bitwise_and(n2, 7)
            gather_start(gbuf, gsem, q2, r2)
            pe_start(n2, pebuf, pesem)

    slab_copy_start(0, 0)
    slab_copy_wait(0)
    gather_start(g0, gs0, 0, 0)
    gather_start(g1, gs1, 0, 1)
    pe_start(0, pe0, ps0)
    pe_start(1, pe1, ps1)

    def loop_body(m, carry):
        block(m * 2, 0)
        block(m * 2 + 1, 1)
        return carry

    lax.fori_loop(0, 4 * NSLAB, loop_body, 0)
    out_wait(o0, os0)
    out_wait(o1, os1)


_lookup = pl.kernel(
    _lookup_body,
    out_type=jax.ShapeDtypeStruct((S, D, B), jnp.float32),
    mesh=plsc.VectorSubcoreMesh(core_axis_name="c", subcore_axis_name="s"),
    scratch_types=[
        pltpu.VMEM((2, 8, BB), jnp.int32),
        pltpu.VMEM((BB, 2 * D), jnp.float32),
        pltpu.VMEM((BB, 2 * D), jnp.float32),
        pltpu.VMEM((D, BB), jnp.float32),
        pltpu.VMEM((D, BB), jnp.float32),
        pltpu.VMEM((D, L), jnp.float32),
        pltpu.VMEM((D, L), jnp.float32),
        pltpu.SemaphoreType.DMA((2,)),
        pltpu.SemaphoreType.DMA,
        pltpu.SemaphoreType.DMA,
        pltpu.SemaphoreType.DMA,
        pltpu.SemaphoreType.DMA,
        pltpu.SemaphoreType.DMA,
        pltpu.SemaphoreType.DMA,
    ],
    compiler_params=pltpu.CompilerParams(
        use_tc_tiling_on_sc=True, needs_layout_passes=False
    ),
)


def kernel(x, table):
    x_t = jnp.transpose(x).astype(jnp.int32)
    tt = jnp.transpose(table)
    t8 = _relayout(tt)
    pe = jnp.asarray(_PE_DIAG)
    out_t = _lookup(t8, x_t, pe)
    return jnp.transpose(out_t, (2, 0, 1))


# speedup vs baseline: 1.1328x; 1.1328x over previous
import numpy as np
import jax
import jax.numpy as jnp
from jax import lax
from jax.experimental import pallas as pl
from jax.experimental.pallas import tpu as pltpu
from jax.experimental.pallas import tpu_sc as plsc

B = 4096
S = 200
D = 64
V = 1000000
NW = 32
BB = 128
L = 16
NSLAB = 25
CS = 256
NFULL = V // CS
TAIL0 = NFULL * CS
TAILW = V - TAIL0


def _positional_encoding_np():
    pos = np.arange(S, dtype=np.float32)[:, None]
    idx = np.arange(D, dtype=np.float32)[None, :]
    angle_rates = 1.0 / np.power(
        np.float32(10000.0), 2.0 * np.floor(idx / 2.0) / np.float32(D)
    )
    angle_rads = (pos * angle_rates).astype(np.float32)
    sines = np.sin(angle_rads[:, 0::2])
    cosines = np.cos(angle_rads[:, 1::2])
    pe = np.concatenate([sines[:, :, None], cosines[:, :, None]], axis=-1)
    return pe.reshape(S, D).astype(np.float32)


_PE = _positional_encoding_np()
_PE_DIAG = np.empty((S, D, L), np.float32)
for _k in range(D // L):
    for _d in range(L):
        for _l in range(L):
            _PE_DIAG[:, L * _k + _d, _l] = _PE[:, L * _k + (_d + _l) % L]


def _relayout_body(tt_hbm, t8_hbm, i0, i1, o0, o1, it, is0, is1, os0, os1):
    wid = lax.axis_index("s") * 2 + lax.axis_index("c")
    iota = lax.iota(jnp.int32, L)

    def in_start(s, ibuf, isem):
        c0 = pl.multiple_of(s * CS, CS)
        pltpu.make_async_copy(tt_hbm.at[:, pl.ds(c0, CS)], ibuf, isem).start()

    def in_wait(ibuf, isem):
        pltpu.make_async_copy(tt_hbm.at[:, pl.ds(0, CS)], ibuf, isem).wait()

    def out_start(s, obuf, osem):
        r0 = pl.multiple_of(s * CS, CS)
        pltpu.make_async_copy(obuf, t8_hbm.at[pl.ds(r0, CS)], osem).start()

    def out_wait(obuf, osem):
        pltpu.make_async_copy(obuf, t8_hbm.at[pl.ds(0, CS)], osem).wait()

    def transpose_slab(ibuf, obuf):
        rows = [iota + g * L for g in range(CS // L)]

        @plsc.parallel_loop(0, L, unroll=4)
        def d_body(d):
            diag = lax.bitwise_and(iota + d, L - 1)
            for k in range(D // L):
                kd = diag + k * L
                for g in range(CS // L):
                    val = plsc.load_gather(ibuf, [kd, rows[g]])
                    plsc.store_scatter(obuf, [rows[g], kd], val * 8.0)

    ibufs, obufs = (i0, i1), (o0, o1)
    isems, osems = (is0, is1), (os0, os1)

    nmine = jnp.int32(NFULL // NW) + jnp.where(wid < NFULL % NW, 1, 0)

    in_start(wid, i0, is0)
    in_start(wid + NW, i1, is1)

    def step(i, carry):
        for half in range(2):
            ibuf, obuf = ibufs[half], obufs[half]
            isem, osem = isems[half], osems[half]
            ii = i * 2 + half
            s = wid + ii * NW

            @pl.when(ii < nmine)
            def _():
                in_wait(ibuf, isem)

                @pl.when(ii >= 2)
                def _():
                    out_wait(obuf, osem)

                transpose_slab(ibuf, obuf)
                out_start(s, obuf, osem)

                @pl.when(ii + 2 < nmine)
                def _():
                    in_start(s + 2 * NW, ibuf, isem)

        return carry

    lax.fori_loop(0, (NFULL // NW + 2) // 2, step, 0)

    @pl.when(nmine >= 1)
    def _():
        out_wait(o0, os0)

    @pl.when(nmine >= 2)
    def _():
        out_wait(o1, os1)

    @pl.when(wid == 0)
    def _():
        pltpu.sync_copy(tt_hbm.at[:, pl.ds(TAIL0, TAILW)], it)
        rows = [lax.iota(jnp.int32, L) + g * L for g in range(TAILW // L)]

        @plsc.parallel_loop(0, L, unroll=4)
        def d_body(d):
            diag = lax.bitwise_and(lax.iota(jnp.int32, L) + d, L - 1)
            for k in range(D // L):
                kd = diag + k * L
                for g in range(TAILW // L):
                    val = plsc.load_gather(it, [kd, rows[g]])
                    plsc.store_scatter(o0, [rows[g], kd], val * 8.0)
        pltpu.sync_copy(o0.at[pl.ds(0, TAILW)], t8_hbm.at[pl.ds(TAIL0, TAILW)])


_relayout = pl.kernel(
    _relayout_body,
    out_type=jax.ShapeDtypeStruct((V, 2 * D), jnp.float32),
    mesh=plsc.VectorSubcoreMesh(core_axis_name="c", subcore_axis_name="s"),
    scratch_types=[
        pltpu.VMEM((D, CS), jnp.float32),
        pltpu.VMEM((D, CS), jnp.float32),
        pltpu.VMEM((CS, 2 * D), jnp.float32),
        pltpu.VMEM((CS, 2 * D), jnp.float32),
        pltpu.VMEM((D, TAILW), jnp.float32),
        pltpu.SemaphoreType.DMA,
        pltpu.SemaphoreType.DMA,
        pltpu.SemaphoreType.DMA,
        pltpu.SemaphoreType.DMA,
    ],
    compiler_params=pltpu.CompilerParams(
        use_tc_tiling_on_sc=True, needs_layout_passes=False
    ),
)


def _lookup_body(t8_hbm, x_hbm, pe_hbm, out_hbm,
                 ixraw, g0, g1, o0, o1, pe0, pe1,
                 ixs, gs0, gs1, os0, os1, ps0, ps1):
    wid = lax.axis_index("s") * 2 + lax.axis_index("c")
    sid0 = wid * NSLAB

    iota = lax.iota(jnp.int32, L)

    def slab_jb(k):
        sid = sid0 + k
        j0 = pl.multiple_of(lax.shift_left(lax.shift_right_logical(sid, 5), 3), 8)
        b0 = pl.multiple_of(lax.shift_left(lax.bitwise_and(sid, 31), 7), BB)
        return j0, b0

    def block_jb(n):
        j0, b0 = slab_jb(lax.shift_right_logical(n, 3))
        return j0 + lax.bitwise_and(n, 7), b0

    def slab_copy_start(k, q):
        j0, b0 = slab_jb(k)
        pltpu.make_async_copy(
            x_hbm.at[pl.ds(j0, 8), pl.ds(b0, BB)], ixraw.at[q], ixs.at[q]
        ).start()

    def slab_copy_wait(q):
        pltpu.make_async_copy(
            x_hbm.at[pl.ds(0, 8), pl.ds(0, BB)], ixraw.at[q], ixs.at[q]
        ).wait()

    def gather_start(gbuf, gsem, q, r):
        pltpu.make_async_copy(t8_hbm.at[ixraw.at[q, r]], gbuf, gsem).start()

    def gather_wait(gbuf, gsem):
        pltpu.make_async_copy(t8_hbm.at[ixraw.at[0, 0]], gbuf, gsem).wait()

    def pe_start(n, pebuf, pesem):
        j, _ = block_jb(n)
        pltpu.make_async_copy(pe_hbm.at[j], pebuf, pesem).start()

    def pe_wait(pebuf, pesem):
        pltpu.make_async_copy(pe_hbm.at[0], pebuf, pesem).wait()

    def out_start(n, obuf, osem):
        j, b0 = block_jb(n)
        pltpu.make_async_copy(
            obuf, out_hbm.at[j, :, pl.ds(b0, BB)], osem
        ).start()

    def out_wait(obuf, osem):
        pltpu.make_async_copy(
            obuf, out_hbm.at[0, :, pl.ds(0, BB)], osem
        ).wait()

    def compute(gbuf, obuf, pebuf):
        rows = [iota + g * L for g in range(BB // L)]

        @plsc.parallel_loop(0, L, unroll=4)
        def d_body(d):
            diag = lax.bitwise_and(iota + d, L - 1)
            for k in range(D // L):
                kd = diag + k * L
                pev = pebuf[k * L + d, :]
                for g in range(BB // L):
                    val = plsc.load_gather(gbuf, [rows[g], kd])
                    plsc.store_scatter(obuf, [kd, rows[g]], val + pev)

    gbufs, obufs, gsems, osems = (g0, g1), (o0, o1), (gs0, gs1), (os0, os1)
    pebufs, pesems = (pe0, pe1), (ps0, ps1)

    def block(n, p):
        gbuf, obuf, gsem, osem = gbufs[p], obufs[p], gsems[p], osems[p]
        pebuf, pesem = pebufs[p], pesems[p]
        k = lax.shift_right_logical(n, 3)
        r = lax.bitwise_and(n, 7)
        qnext = lax.bitwise_and(k + 1, 1)

        @pl.when(jnp.logical_and(r == 0, k + 1 < NSLAB))
        def _():
            slab_copy_start(k + 1, qnext)

        gather_wait(gbuf, gsem)
        pe_wait(pebuf, pesem)

        @pl.when(n >= 2)
        def _():
            out_wait(obuf, osem)

        compute(gbuf, obuf, pebuf)
        out_start(n, obuf, osem)

        @pl.when(jnp.logical_and(r == 6, k + 1 < NSLAB))
        def _():
            slab_copy_wait(qnext)

        @pl.when(n + 2 < 8 * NSLAB)
        def _():
            n2 = n + 2
            q2 = lax.bitwise_and(lax.shift_right_logical(n2, 3), 1)
            r2 = lax.bitwise_and(n2, 7)
            gather_start(gbuf, gsem, q2, r2)
            pe_start(n2, pebuf, pesem)

    slab_copy_start(0, 0)
    slab_copy_wait(0)
    gather_start(g0, gs0, 0, 0)
    gather_start(g1, gs1, 0, 1)
    pe_start(0, pe0, ps0)
    pe_start(1, pe1, ps1)

    def loop_body(m, carry):
        block(m * 2, 0)
        block(m * 2 + 1, 1)
        return carry

    lax.fori_loop(0, 4 * NSLAB, loop_body, 0)
    out_wait(o0, os0)
    out_wait(o1, os1)


_lookup = pl.kernel(
    _lookup_body,
    out_type=jax.ShapeDtypeStruct((S, D, B), jnp.float32),
    mesh=plsc.VectorSubcoreMesh(core_axis_name="c", subcore_axis_name="s"),
    scratch_types=[
        pltpu.VMEM((2, 8, BB), jnp.int32),
        pltpu.VMEM((BB, 2 * D), jnp.float32),
        pltpu.VMEM((BB, 2 * D), jnp.float32),
        pltpu.VMEM((D, BB), jnp.float32),
        pltpu.VMEM((D, BB), jnp.float32),
        pltpu.VMEM((D, L), jnp.float32),
        pltpu.VMEM((D, L), jnp.float32),
        pltpu.SemaphoreType.DMA((2,)),
        pltpu.SemaphoreType.DMA,
        pltpu.SemaphoreType.DMA,
        pltpu.SemaphoreType.DMA,
        pltpu.SemaphoreType.DMA,
        pltpu.SemaphoreType.DMA,
        pltpu.SemaphoreType.DMA,
    ],
    compiler_params=pltpu.CompilerParams(
        use_tc_tiling_on_sc=True, needs_layout_passes=False
    ),
)


def kernel(x, table):
    x_t = jnp.transpose(x).astype(jnp.int32)
    tt = jnp.transpose(table)
    t8 = _relayout(tt)
    pe = jnp.asarray(_PE_DIAG)
    out_t = _lookup(t8, x_t, pe)
    return jnp.transpose(out_t, (2, 0, 1))


# speedup vs baseline: 1.2029x; 1.0619x over previous
import numpy as np
import jax
import jax.numpy as jnp
from jax import lax
from jax.experimental import pallas as pl
from jax.experimental.pallas import tpu as pltpu
from jax.experimental.pallas import tpu_sc as plsc

B = 4096
S = 200
D = 64
V = 1000000
NW = 32
BB = 128
L = 16
NSLAB = 25
CS = 256
NFULL = V // CS
TAIL0 = NFULL * CS
TAILW = V - TAIL0


def _positional_encoding_np():
    pos = np.arange(S, dtype=np.float32)[:, None]
    idx = np.arange(D, dtype=np.float32)[None, :]
    angle_rates = 1.0 / np.power(
        np.float32(10000.0), 2.0 * np.floor(idx / 2.0) / np.float32(D)
    )
    angle_rads = (pos * angle_rates).astype(np.float32)
    sines = np.sin(angle_rads[:, 0::2])
    cosines = np.cos(angle_rads[:, 1::2])
    pe = np.concatenate([sines[:, :, None], cosines[:, :, None]], axis=-1)
    return pe.reshape(S, D).astype(np.float32)


_PE = _positional_encoding_np()
_PE_DIAG = np.empty((S, D, L), np.float32)
for _k in range(D // L):
    for _d in range(L):
        for _l in range(L):
            _PE_DIAG[:, L * _k + _d, _l] = _PE[:, L * _k + (_d + _l) % L]


def _relayout_body(tt_hbm, t8_hbm, i0, i1, o0, o1, it, is0, is1, os0, os1):
    wid = lax.axis_index("s") * 2 + lax.axis_index("c")
    iota = lax.iota(jnp.int32, L)

    def in_start(s, ibuf, isem):
        c0 = pl.multiple_of(s * CS, CS)
        pltpu.make_async_copy(tt_hbm.at[:, pl.ds(c0, CS)], ibuf, isem).start()

    def in_wait(ibuf, isem):
        pltpu.make_async_copy(tt_hbm.at[:, pl.ds(0, CS)], ibuf, isem).wait()

    def out_start(s, obuf, osem):
        r0 = pl.multiple_of(s * CS, CS)
        pltpu.make_async_copy(obuf, t8_hbm.at[pl.ds(r0, CS)], osem).start()

    def out_wait(obuf, osem):
        pltpu.make_async_copy(obuf, t8_hbm.at[pl.ds(0, CS)], osem).wait()

    def transpose_slab(ibuf, obuf):
        rows = [iota + g * L for g in range(CS // L)]

        @plsc.parallel_loop(0, L, unroll=4)
        def d_body(d):
            diag = lax.bitwise_and(iota + d, L - 1)
            for k in range(D // L):
                kd = diag + k * L
                for g in range(CS // L):
                    val = plsc.load_gather(ibuf, [kd, rows[g]])
                    plsc.store_scatter(obuf, [rows[g], kd], val * 8.0)

    ibufs, obufs = (i0, i1), (o0, o1)
    isems, osems = (is0, is1), (os0, os1)

    nmine = jnp.int32(NFULL // NW) + jnp.where(wid < NFULL % NW, 1, 0)

    in_start(wid, i0, is0)
    in_start(wid + NW, i1, is1)

    def step(i, carry):
        for half in range(2):
            ibuf, obuf = ibufs[half], obufs[half]
            isem, osem = isems[half], osems[half]
            ii = i * 2 + half
            s = wid + ii * NW

            @pl.when(ii < nmine)
            def _():
                in_wait(ibuf, isem)

                @pl.when(ii >= 2)
                def _():
                    out_wait(obuf, osem)

                transpose_slab(ibuf, obuf)
                out_start(s, obuf, osem)

                @pl.when(ii + 2 < nmine)
                def _():
                    in_start(s + 2 * NW, ibuf, isem)

        return carry

    lax.fori_loop(0, (NFULL // NW + 2) // 2, step, 0)

    @pl.when(nmine >= 1)
    def _():
        out_wait(o0, os0)

    @pl.when(nmine >= 2)
    def _():
        out_wait(o1, os1)

    @pl.when(wid == 0)
    def _():
        pltpu.sync_copy(tt_hbm.at[:, pl.ds(TAIL0, TAILW)], it)
        rows = [lax.iota(jnp.int32, L) + g * L for g in range(TAILW // L)]

        @plsc.parallel_loop(0, L, unroll=4)
        def d_body(d):
            diag = lax.bitwise_and(lax.iota(jnp.int32, L) + d, L - 1)
            for k in range(D // L):
                kd = diag + k * L
                for g in range(TAILW // L):
                    val = plsc.load_gather(it, [kd, rows[g]])
                    plsc.store_scatter(o0, [rows[g], kd], val * 8.0)
        pltpu.sync_copy(o0.at[pl.ds(0, TAILW)], t8_hbm.at[pl.ds(TAIL0, TAILW)])


_relayout = pl.kernel(
    _relayout_body,
    out_type=jax.ShapeDtypeStruct((V, 2 * D), jnp.float32),
    mesh=plsc.VectorSubcoreMesh(core_axis_name="c", subcore_axis_name="s"),
    scratch_types=[
        pltpu.VMEM((D, CS), jnp.float32),
        pltpu.VMEM((D, CS), jnp.float32),
        pltpu.VMEM((CS, 2 * D), jnp.float32),
        pltpu.VMEM((CS, 2 * D), jnp.float32),
        pltpu.VMEM((D, TAILW), jnp.float32),
        pltpu.SemaphoreType.DMA,
        pltpu.SemaphoreType.DMA,
        pltpu.SemaphoreType.DMA,
        pltpu.SemaphoreType.DMA,
    ],
    compiler_params=pltpu.CompilerParams(
        use_tc_tiling_on_sc=True, needs_layout_passes=False
    ),
)


def _lookup_body(t8_hbm, x_hbm, pe_hbm, out_hbm,
                 ixraw, g0, g1, o0, o1, pe_all,
                 ixs, gs0, gs1, os0, os1):
    wid = lax.axis_index("s") * 2 + lax.axis_index("c")
    sid0 = wid * NSLAB

    iota = lax.iota(jnp.int32, L)

    jbase = pl.multiple_of(
        lax.shift_left(lax.shift_right_logical(sid0, 5), 3), 8)
    pltpu.sync_copy(pe_hbm.at[pl.ds(jbase, 2 * 8)], pe_all)

    def slab_jb(k):
        sid = sid0 + k
        j0 = pl.multiple_of(lax.shift_left(lax.shift_right_logical(sid, 5), 3), 8)
        b0 = pl.multiple_of(lax.shift_left(lax.bitwise_and(sid, 31), 7), BB)
        return j0, b0

    def block_jb(n):
        j0, b0 = slab_jb(lax.shift_right_logical(n, 3))
        return j0 + lax.bitwise_and(n, 7), b0

    def slab_copy_start(k, q):
        j0, b0 = slab_jb(k)
        pltpu.make_async_copy(
            x_hbm.at[pl.ds(j0, 8), pl.ds(b0, BB)], ixraw.at[q], ixs.at[q]
        ).start()

    def slab_copy_wait(q):
        pltpu.make_async_copy(
            x_hbm.at[pl.ds(0, 8), pl.ds(0, BB)], ixraw.at[q], ixs.at[q]
        ).wait()

    def gather_start(gbuf, gsem, q, r):
        pltpu.make_async_copy(t8_hbm.at[ixraw.at[q, r]], gbuf, gsem).start()

    def gather_wait(gbuf, gsem):
        pltpu.make_async_copy(t8_hbm.at[ixraw.at[0, 0]], gbuf, gsem).wait()

    def out_start(n, obuf, osem):
        j, b0 = block_jb(n)
        pltpu.make_async_copy(
            obuf, out_hbm.at[j, :, pl.ds(b0, BB)], osem
        ).start()

    def out_wait(obuf, osem):
        pltpu.make_async_copy(
            obuf, out_hbm.at[0, :, pl.ds(0, BB)], osem
        ).wait()

    def compute(gbuf, obuf, jj):
        rows = [iota + g * L for g in range(BB // L)]

        @plsc.parallel_loop(0, L, unroll=4)
        def d_body(d):
            diag = lax.bitwise_and(iota + d, L - 1)
            for k in range(D // L):
                kd = diag + k * L
                pev = pe_all[jj, pl.ds((k * L + d) * L, L)]
                for g in range(BB // L):
                    val = plsc.load_gather(gbuf, [rows[g], kd])
                    plsc.store_scatter(obuf, [kd, rows[g]], val + pev)

    gbufs, obufs, gsems, osems = (g0, g1), (o0, o1), (gs0, gs1), (os0, os1)

    def block(n, p):
        gbuf, obuf, gsem, osem = gbufs[p], obufs[p], gsems[p], osems[p]
        k = lax.shift_right_logical(n, 3)
        r = lax.bitwise_and(n, 7)
        qnext = lax.bitwise_and(k + 1, 1)

        @pl.when(jnp.logical_and(r == 0, k + 1 < NSLAB))
        def _():
            slab_copy_start(k + 1, qnext)

        gather_wait(gbuf, gsem)

        @pl.when(n >= 2)
        def _():
            out_wait(obuf, osem)

        j, _ = block_jb(n)
        compute(gbuf, obuf, j - jbase)
        out_start(n, obuf, osem)

        @pl.when(jnp.logical_and(r == 6, k + 1 < NSLAB))
        def _():
            slab_copy_wait(qnext)

        @pl.when(n + 2 < 8 * NSLAB)
        def _():
            n2 = n + 2
            q2 = lax.bitwise_and(lax.shift_right_logical(n2, 3), 1)
            r2 = lax.bitwise_and(n2, 7)
            gather_start(gbuf, gsem, q2, r2)

    slab_copy_start(0, 0)
    slab_copy_wait(0)
    gather_start(g0, gs0, 0, 0)
    gather_start(g1, gs1, 0, 1)

    def loop_body(m, carry):
        block(m * 2, 0)
        block(m * 2 + 1, 1)
        return carry

    lax.fori_loop(0, 4 * NSLAB, loop_body, 0)
    out_wait(o0, os0)
    out_wait(o1, os1)


_lookup = pl.kernel(
    _lookup_body,
    out_type=jax.ShapeDtypeStruct((S, D, B), jnp.float32),
    mesh=plsc.VectorSubcoreMesh(core_axis_name="c", subcore_axis_name="s"),
    scratch_types=[
        pltpu.VMEM((2, 8, BB), jnp.int32),
        pltpu.VMEM((BB, 2 * D), jnp.float32),
        pltpu.VMEM((BB, 2 * D), jnp.float32),
        pltpu.VMEM((D, BB), jnp.float32),
        pltpu.VMEM((D, BB), jnp.float32),
        pltpu.VMEM((2 * 8, D * L), jnp.float32),
        pltpu.SemaphoreType.DMA((2,)),
        pltpu.SemaphoreType.DMA,
        pltpu.SemaphoreType.DMA,
        pltpu.SemaphoreType.DMA,
        pltpu.SemaphoreType.DMA,
    ],
    compiler_params=pltpu.CompilerParams(
        use_tc_tiling_on_sc=True, needs_layout_passes=False
    ),
)


def kernel(x, table):
    x_t = jnp.transpose(x).astype(jnp.int32)
    tt = jnp.transpose(table)
    t8 = _relayout(tt)
    pe = jnp.asarray(_PE_DIAG.reshape(S, D * L))
    out_t = _lookup(t8, x_t, pe)
    return jnp.transpose(out_t, (2, 0, 1))
